# Initial kernel scaffold; baseline (speedup 1.0000x reference)
import jax
import jax.numpy as jnp
from jax.experimental import pallas as pl


def kernel(x, edge_index, edge_weight, Wz, bz, Wlz, blz, Wr, br, Wlr, blr, Wh, bh, Wlh, blh, Wout, bout):
    raise NotImplementedError("write your pallas kernel here")


# speedup vs baseline: 29.0788x; 29.0788x over previous
import functools

import jax
import jax.numpy as jnp
from jax import lax
from jax.experimental import pallas as pl
from jax.experimental.pallas import tpu as pltpu
from jax.experimental.pallas import tpu_sc as plsc

NC = 2
NS = 16
LANES = 16


def _sc_propagate(x, src, dst, w):
  N, F = x.shape
  E = src.shape[0]
  NPAD = ((N + NS * LANES - 1) // (NS * LANES)) * NS * LANES
  NPT = NPAD // NS
  XPT = ((N + NS * 8 - 1) // (NS * 8)) * 8
  NXP = XPT * NS
  EC = E // NC
  ET = EC // NS
  CH = 128
  NFULL = ET // CH
  REM = ET - NFULL * CH
  DEG_PT = E // NS
  DCH = 2000
  NDCH = DEG_PT // DCH

  mesh = plsc.VectorSubcoreMesh(core_axis_name="c", subcore_axis_name="s")

  NPR = NPAD // F
  RPT = NPR // NS

  scratch_types = [
      pltpu.VMEM_SHARED((NPR, F), jnp.float32),
      pltpu.VMEM_SHARED((NPAD,), jnp.float32),
      pltpu.VMEM_SHARED((NXP, F), jnp.float32),
      pltpu.VMEM((NPR, F), jnp.float32),
      pltpu.VMEM((RPT, F), jnp.float32),
      pltpu.VMEM((NPT,), jnp.float32),
      pltpu.VMEM((NPAD,), jnp.float32),
      pltpu.VMEM((NPR,), jnp.int32),
      pltpu.VMEM((DCH,), jnp.int32),
      pltpu.VMEM((DCH,), jnp.float32),
      pltpu.VMEM((CH,), jnp.int32),
      pltpu.VMEM((CH,), jnp.int32),
      pltpu.VMEM((CH,), jnp.float32),
      pltpu.VMEM((CH,), jnp.float32),
      pltpu.VMEM((CH, F), jnp.float32),
      pltpu.SemaphoreType.DMA,
  ]

  @functools.partial(
      pl.kernel,
      out_type=(
          jax.ShapeDtypeStruct((NC, NXP, F), jnp.float32),
          jax.ShapeDtypeStruct((NPAD,), jnp.float32),
      ),
      mesh=mesh,
      compiler_params=pltpu.CompilerParams(needs_layout_passes=False),
      scratch_types=scratch_types,
  )
  def body(x_h, src_h, dst_h, w_h, xpp_h, dinv_h,
           sdeg, sdinv, sxp, ldeg, dbuf, dacc, dlocal, riota,
           didxL, dwL, sidx, didx, wbuf, nbuf, rows, sem):
    c = lax.axis_index("c")
    s = lax.axis_index("s")
    zero16 = jnp.zeros((LANES,), jnp.float32)

    @pl.loop(0, CH)
    def _(r):
      for j in range(F // LANES):
        rows[r, pl.ds(j * LANES, LANES)] = zero16

    @pl.loop(0, NPR)
    def _(r):
      for j in range(F // LANES):
        ldeg[r, pl.ds(j * LANES, LANES)] = zero16

    pltpu.sync_copy(rows.at[pl.ds(0, RPT)], sdeg.at[pl.ds(s * RPT, RPT)])

    @pl.loop(0, NPR // LANES)
    def _(i):
      riota[pl.ds(i * LANES, LANES)] = \
          lax.iota(jnp.int32, LANES) + i * LANES

    @pl.loop(0, NDCH)
    def _(k):
      base = s * DEG_PT + k * DCH
      pltpu.sync_copy(dst_h.at[pl.ds(base, DCH)], didxL)
      pltpu.sync_copy(w_h.at[pl.ds(base, DCH)], dwL)

      @pl.loop(0, DCH // LANES)
      def _(i):
        dv = didxL[pl.ds(i * LANES, LANES)]
        wv = dwL[pl.ds(i * LANES, LANES)]
        plsc.addupdate_scatter(ldeg, [dv >> 7, dv & 127], wv)

    plsc.subcore_barrier()
    pltpu.sync_copy(ldeg, sdeg.at[riota], add=True)
    plsc.subcore_barrier()

    pltpu.sync_copy(sdeg.at[pl.ds(s * RPT, RPT)], dbuf)

    @pl.loop(0, RPT)
    def _(r):
      for j in range(F // LANES):
        v = dbuf[r, pl.ds(j * LANES, LANES)] + 1.0
        bi = plsc.bitcast(v, jnp.int32)
        y = plsc.bitcast(jnp.int32(0x5F3759DF) - (bi >> 1), jnp.float32)
        y = y * (1.5 - 0.5 * v * y * y)
        y = y * (1.5 - 0.5 * v * y * y)
        y = y * (1.5 - 0.5 * v * y * y)
        dacc[pl.ds(r * F + j * LANES, LANES)] = y

    pltpu.sync_copy(dacc, sdinv.at[pl.ds(s * NPT, NPT)])

    @pl.when(c == 0)
    def _():
      pltpu.sync_copy(dacc, dinv_h.at[pl.ds(s * NPT, NPT)])

    plsc.subcore_barrier()
    pltpu.sync_copy(sdinv, dlocal)

    r0 = s * XPT
    for kk in range(XPT // CH):
      pltpu.sync_copy(rows, sxp.at[pl.ds(r0 + kk * CH, CH)])
    if XPT % CH:
      pltpu.sync_copy(rows.at[pl.ds(0, XPT % CH)],
                      sxp.at[pl.ds(r0 + (XPT // CH) * CH, XPT % CH)])
    plsc.subcore_barrier()

    def chunk_body():
      pltpu.async_copy(x_h.at[sidx], rows, sem).wait()

      @pl.loop(0, CH // LANES)
      def _(i):
        sl = pl.ds(i * LANES, LANES)
        sv = sidx[sl]
        dv = didx[sl]
        wv = wbuf[sl]
        nbuf[sl] = plsc.load_gather(dlocal, [sv]) * wv * \
            plsc.load_gather(dlocal, [dv])

      @pl.loop(0, CH // LANES)
      def _(g):
        nv = nbuf[pl.ds(g * LANES, LANES)]
        for l in range(LANES):
          nval = nv[l]
          r = g * LANES + l
          for j in range(F // LANES):
            sl = pl.ds(j * LANES, LANES)
            rows[r, sl] = rows[r, sl] * nval

      pltpu.sync_copy(rows, sxp.at[didx], add=True)

    eb = c * EC + s * ET

    @pl.loop(0, NFULL)
    def _(k):
      eo = eb + k * CH
      pltpu.sync_copy(src_h.at[pl.ds(eo, CH)], sidx)
      pltpu.sync_copy(dst_h.at[pl.ds(eo, CH)], didx)
      pltpu.sync_copy(w_h.at[pl.ds(eo, CH)], wbuf)
      chunk_body()

    if REM:
      eo = eb + NFULL * CH
      pltpu.sync_copy(src_h.at[pl.ds(eo, REM)], sidx.at[pl.ds(0, REM)])
      pltpu.sync_copy(dst_h.at[pl.ds(eo, REM)], didx.at[pl.ds(0, REM)])
      pltpu.sync_copy(w_h.at[pl.ds(eo, REM)], wbuf.at[pl.ds(0, REM)])
      for i in range(REM // LANES, CH // LANES):
        wbuf[pl.ds(i * LANES, LANES)] = zero16
      chunk_body()

    plsc.subcore_barrier()

    rr = s * XPT
    pltpu.sync_copy(sxp.at[pl.ds(rr, XPT)], xpp_h.at[c, pl.ds(rr, XPT)])

  return body(x, src, dst, w)


def _tc_head(xpp, x, dinv_n, Wz, bz, Wlz, blz, Wh, bh, Wlh, blh, Wout, bout):
  N, F = x.shape
  B = 1000
  NB = N // B

  def body(xpp_ref, x_ref, dinv_ref, wz_ref, bz_ref, wlz_ref, blz_ref,
           wh_ref, bh_ref, wlh_ref, blh_ref, wout_ref, bout_ref, y_ref,
           mz_s, cz_s, mh_s, ch_s):
    @pl.when(pl.program_id(0) == 0)
    def _():
      wlz_t = wlz_ref[0:F, :]
      wlh_t = wlh_ref[0:F, :]
      mz_s[...] = jnp.dot(wz_ref[...], wlz_t, preferred_element_type=jnp.float32)
      cz_s[...] = jnp.dot(bz_ref[...], wlz_t, preferred_element_type=jnp.float32) + blz_ref[...]
      mh_s[...] = jnp.dot(wh_ref[...], wlh_t, preferred_element_type=jnp.float32)
      ch_s[...] = jnp.dot(bh_ref[...], wlh_t, preferred_element_type=jnp.float32) + blh_ref[...]

    d = dinv_ref[...]
    xp = xpp_ref[0] + xpp_ref[1] + d * d * x_ref[...]
    z = jax.nn.sigmoid(jnp.dot(xp, mz_s[...], preferred_element_type=jnp.float32) + cz_s[...])
    ht = jnp.tanh(jnp.dot(xp, mh_s[...], preferred_element_type=jnp.float32) + ch_s[...])
    h = (1.0 - z) * ht
    y_ref[...] = jnp.dot(jnp.maximum(h, 0.0), wout_ref[...],
                         preferred_element_type=jnp.float32) + bout_ref[...]

  full = lambda shape: pl.BlockSpec(shape, lambda i: (0,) * len(shape))
  return pl.pallas_call(
      body,
      grid=(NB,),
      in_specs=[
          pl.BlockSpec((NC, B, F), lambda i: (0, i, 0)),
          pl.BlockSpec((B, F), lambda i: (i, 0)),
          pl.BlockSpec((B, 1), lambda i: (i, 0)),
          full((F, F)), full((1, F)), full((2 * F, F)), full((1, F)),
          full((F, F)), full((1, F)), full((2 * F, F)), full((1, F)),
          full((F, 1)), full((1, 1)),
      ],
      out_specs=pl.BlockSpec((B, 1), lambda i: (i, 0)),
      out_shape=jax.ShapeDtypeStruct((N, 1), jnp.float32),
      scratch_shapes=[
          pltpu.VMEM((F, F), jnp.float32),
          pltpu.VMEM((1, F), jnp.float32),
          pltpu.VMEM((F, F), jnp.float32),
          pltpu.VMEM((1, F), jnp.float32),
      ],
  )(xpp, x, dinv_n, Wz, bz, Wlz, blz, Wh, bh, Wlh, blh, Wout, bout)


def kernel(x, edge_index, edge_weight, Wz, bz, Wlz, blz, Wr, br, Wlr, blr,
           Wh, bh, Wlh, blh, Wout, bout):
  N, F = x.shape
  src = edge_index[0]
  dst = edge_index[1]
  xpp, dinv_pad = _sc_propagate(x, src, dst, edge_weight)
  xpp = xpp[:, :N]
  dinv_n = dinv_pad[:N].reshape(N, 1)
  return _tc_head(xpp, x, dinv_n,
                  Wz, bz.reshape(1, F), Wlz, blz.reshape(1, F),
                  Wh, bh.reshape(1, F), Wlh, blh.reshape(1, F),
                  Wout, bout.reshape(1, 1))
